# Initial kernel scaffold; baseline (speedup 1.0000x reference)
import jax
import jax.numpy as jnp
from jax.experimental import pallas as pl


def kernel(x, W, b):
    raise NotImplementedError("write your pallas kernel here")


# speedup vs baseline: 1.4228x; 1.4228x over previous
import functools

import jax
import jax.numpy as jnp
from jax import lax
from jax.experimental import pallas as pl

N_TOKENS = 32768
DIM_IN = 4096
NUM_EXPERTS = 64
BT = 512


def _router_block(x_ref, wt_ref, b_ref, vals_ref, idx_ref):
    logits = jnp.dot(x_ref[...], wt_ref[...], preferred_element_type=jnp.float32)
    logits = logits + b_ref[...]
    iota = lax.broadcasted_iota(jnp.int32, logits.shape, 1)
    max1 = jnp.max(logits, axis=1, keepdims=True)
    idx1 = jnp.min(jnp.where(logits == max1, iota, NUM_EXPERTS), axis=1, keepdims=True)
    masked = jnp.where(iota == idx1, -jnp.inf, logits)
    max2 = jnp.max(masked, axis=1, keepdims=True)
    idx2 = jnp.min(jnp.where(masked == max2, iota, NUM_EXPERTS), axis=1, keepdims=True)
    vals_ref[...] = jnp.concatenate([max1, max2], axis=1)
    idx_ref[...] = jnp.concatenate([idx1, idx2], axis=1)


@jax.jit
def kernel(x, W, b):
    wt = W.T
    b2 = b.reshape(1, NUM_EXPERTS)
    grid = (N_TOKENS // BT,)
    vals, idx = pl.pallas_call(
        _router_block,
        grid=grid,
        in_specs=[
            pl.BlockSpec((BT, DIM_IN), lambda i: (i, 0)),
            pl.BlockSpec((DIM_IN, NUM_EXPERTS), lambda i: (0, 0)),
            pl.BlockSpec((1, NUM_EXPERTS), lambda i: (0, 0)),
        ],
        out_specs=[
            pl.BlockSpec((BT, 2), lambda i: (i, 0)),
            pl.BlockSpec((BT, 2), lambda i: (i, 0)),
        ],
        out_shape=[
            jax.ShapeDtypeStruct((N_TOKENS, 2), jnp.float32),
            jax.ShapeDtypeStruct((N_TOKENS, 2), jnp.int32),
        ],
    )(x, wt, b2)
    return (vals, idx)


# speedup vs baseline: 1.5130x; 1.0634x over previous
import functools

import jax
import jax.numpy as jnp
from jax import lax
from jax.experimental import pallas as pl
from jax.experimental.pallas import tpu as pltpu
from jax.experimental.pallas import tpu_sc as plsc

N_TOKENS = 32768
DIM_IN = 4096
NUM_EXPERTS = 64
BT = 1024

NUM_WORKERS = 32
TOK_PER_W = N_TOKENS // NUM_WORKERS
LANES = 16
NUM_GROUPS = TOK_PER_W // LANES


def _gate_block(x_ref, w_ref, b_ref, out_ref):
    logits_t = lax.dot_general(
        w_ref[...], x_ref[...],
        dimension_numbers=(((1,), (1,)), ((), ())),
        preferred_element_type=jnp.float32,
    )
    out_ref[...] = (logits_t + b_ref[...])[None]


def _sc_top2(logits_hbm, v1_hbm, v2_hbm, i1_hbm, i2_hbm,
             lg_v, v1_v, v2_v, i1_v, i2_v):
    wid = lax.axis_index("s") * 2 + lax.axis_index("c")
    base = wid * TOK_PER_W
    pltpu.sync_copy(logits_hbm.at[wid], lg_v)

    def g_body(g, carry):
        sl = pl.ds(g * LANES, LANES)
        m1 = jnp.full((LANES,), -jnp.inf, jnp.float32)
        m2 = jnp.full((LANES,), -jnp.inf, jnp.float32)
        i1 = jnp.zeros((LANES,), jnp.int32)
        i2 = jnp.zeros((LANES,), jnp.int32)
        for e in range(NUM_EXPERTS):
            v = lg_v[e, sl]
            e_s = jnp.full((LANES,), e, jnp.int32)
            gt1 = v > m1
            gt2 = v > m2
            m2 = jnp.where(gt1, m1, jnp.where(gt2, v, m2))
            i2 = jnp.where(gt1, i1, jnp.where(gt2, e_s, i2))
            m1 = jnp.where(gt1, v, m1)
            i1 = jnp.where(gt1, e_s, i1)
        v1_v[sl] = m1
        v2_v[sl] = m2
        i1_v[sl] = i1
        i2_v[sl] = i2
        return carry

    lax.fori_loop(0, NUM_GROUPS, g_body, 0)
    pltpu.sync_copy(v1_v, v1_hbm.at[pl.ds(base, TOK_PER_W)])
    pltpu.sync_copy(v2_v, v2_hbm.at[pl.ds(base, TOK_PER_W)])
    pltpu.sync_copy(i1_v, i1_hbm.at[pl.ds(base, TOK_PER_W)])
    pltpu.sync_copy(i2_v, i2_hbm.at[pl.ds(base, TOK_PER_W)])


@jax.jit
def kernel(x, W, b):
    b_col = b.reshape(NUM_EXPERTS, 1)
    logits = pl.pallas_call(
        _gate_block,
        grid=(N_TOKENS // BT,),
        in_specs=[
            pl.BlockSpec((BT, DIM_IN), lambda i: (i, 0)),
            pl.BlockSpec((NUM_EXPERTS, DIM_IN), lambda i: (0, 0)),
            pl.BlockSpec((NUM_EXPERTS, 1), lambda i: (0, 0)),
        ],
        out_specs=pl.BlockSpec((1, NUM_EXPERTS, BT), lambda i: (i, 0, 0)),
        out_shape=jax.ShapeDtypeStruct((NUM_WORKERS, NUM_EXPERTS, TOK_PER_W),
                                       jnp.float32),
    )(x, W, b_col)

    sc_call = functools.partial(
        pl.kernel,
        mesh=plsc.VectorSubcoreMesh(core_axis_name="c", subcore_axis_name="s"),
        out_type=[
            jax.ShapeDtypeStruct((N_TOKENS,), jnp.float32),
            jax.ShapeDtypeStruct((N_TOKENS,), jnp.float32),
            jax.ShapeDtypeStruct((N_TOKENS,), jnp.int32),
            jax.ShapeDtypeStruct((N_TOKENS,), jnp.int32),
        ],
        scratch_types=[
            pltpu.VMEM((NUM_EXPERTS, TOK_PER_W), jnp.float32),
            pltpu.VMEM((TOK_PER_W,), jnp.float32),
            pltpu.VMEM((TOK_PER_W,), jnp.float32),
            pltpu.VMEM((TOK_PER_W,), jnp.int32),
            pltpu.VMEM((TOK_PER_W,), jnp.int32),
        ],
    )(_sc_top2)
    v1, v2, i1, i2 = sc_call(logits)
    vals = jnp.stack([v1, v2], axis=1)
    idx = jnp.stack([i1, i2], axis=1)
    return (vals, idx)


# speedup vs baseline: 1.5132x; 1.0001x over previous
import functools

import jax
import jax.numpy as jnp
from jax import lax
from jax.experimental import pallas as pl
from jax.experimental.pallas import tpu as pltpu
from jax.experimental.pallas import tpu_sc as plsc

N_TOKENS = 32768
DIM_IN = 4096
NUM_EXPERTS = 64
BT = 1024
HBT = BT // 2

NUM_WORKERS = 32
TOK_PER_W = N_TOKENS // NUM_WORKERS
LANES = 16
NUM_GROUPS = TOK_PER_W // LANES


def _gate_block(x0_ref, x1_ref, w_ref, b_ref, out_ref):
    lt0 = lax.dot_general(
        w_ref[...], x0_ref[...],
        dimension_numbers=(((1,), (1,)), ((), ())),
        preferred_element_type=jnp.float32,
    )
    lt1 = lax.dot_general(
        w_ref[...], x1_ref[...],
        dimension_numbers=(((1,), (1,)), ((), ())),
        preferred_element_type=jnp.float32,
    )
    logits_t = jnp.concatenate([lt0, lt1], axis=1) + b_ref[...]
    out_ref[...] = jnp.stack(
        [logits_t[:, j * TOK_PER_W:(j + 1) * TOK_PER_W]
         for j in range(BT // TOK_PER_W)], axis=0)


def _sc_top2(logits_hbm, v1_hbm, v2_hbm, i1_hbm, i2_hbm,
             lg_v, v1_v, v2_v, i1_v, i2_v):
    wid = lax.axis_index("s") * 2 + lax.axis_index("c")
    base = wid * TOK_PER_W
    pltpu.sync_copy(logits_hbm.at[wid], lg_v)

    def g_body(g, carry):
        sl = pl.ds(g * LANES, LANES)
        m1 = jnp.full((LANES,), -jnp.inf, jnp.float32)
        m2 = jnp.full((LANES,), -jnp.inf, jnp.float32)
        i1 = jnp.zeros((LANES,), jnp.int32)
        i2 = jnp.zeros((LANES,), jnp.int32)
        for e in range(NUM_EXPERTS):
            v = lg_v[e, sl]
            e_s = jnp.full((LANES,), e, jnp.int32)
            gt1 = v > m1
            gt2 = v > m2
            m2 = jnp.where(gt1, m1, jnp.where(gt2, v, m2))
            i2 = jnp.where(gt1, i1, jnp.where(gt2, e_s, i2))
            m1 = jnp.where(gt1, v, m1)
            i1 = jnp.where(gt1, e_s, i1)
        v1_v[sl] = m1
        v2_v[sl] = m2
        i1_v[sl] = i1
        i2_v[sl] = i2
        return carry

    lax.fori_loop(0, NUM_GROUPS, g_body, 0)
    pltpu.sync_copy(v1_v, v1_hbm.at[pl.ds(base, TOK_PER_W)])
    pltpu.sync_copy(v2_v, v2_hbm.at[pl.ds(base, TOK_PER_W)])
    pltpu.sync_copy(i1_v, i1_hbm.at[pl.ds(base, TOK_PER_W)])
    pltpu.sync_copy(i2_v, i2_hbm.at[pl.ds(base, TOK_PER_W)])


@jax.jit
def kernel(x, W, b):
    b_col = b.reshape(NUM_EXPERTS, 1)
    logits = pl.pallas_call(
        _gate_block,
        grid=(N_TOKENS // BT,),
        in_specs=[
            pl.BlockSpec((HBT, DIM_IN), lambda i: (2 * i, 0)),
            pl.BlockSpec((HBT, DIM_IN), lambda i: (2 * i + 1, 0)),
            pl.BlockSpec((NUM_EXPERTS, DIM_IN), lambda i: (0, 0)),
            pl.BlockSpec((NUM_EXPERTS, 1), lambda i: (0, 0)),
        ],
        out_specs=pl.BlockSpec((BT // TOK_PER_W, NUM_EXPERTS, TOK_PER_W),
                               lambda i: (i, 0, 0)),
        out_shape=jax.ShapeDtypeStruct((NUM_WORKERS, NUM_EXPERTS, TOK_PER_W),
                                       jnp.float32),
        compiler_params=pltpu.CompilerParams(
            vmem_limit_bytes=120 * 1024 * 1024,
        ),
    )(x, x, W, b_col)

    sc_call = functools.partial(
        pl.kernel,
        mesh=plsc.VectorSubcoreMesh(core_axis_name="c", subcore_axis_name="s"),
        out_type=[
            jax.ShapeDtypeStruct((N_TOKENS,), jnp.float32),
            jax.ShapeDtypeStruct((N_TOKENS,), jnp.float32),
            jax.ShapeDtypeStruct((N_TOKENS,), jnp.int32),
            jax.ShapeDtypeStruct((N_TOKENS,), jnp.int32),
        ],
        scratch_types=[
            pltpu.VMEM((NUM_EXPERTS, TOK_PER_W), jnp.float32),
            pltpu.VMEM((TOK_PER_W,), jnp.float32),
            pltpu.VMEM((TOK_PER_W,), jnp.float32),
            pltpu.VMEM((TOK_PER_W,), jnp.int32),
            pltpu.VMEM((TOK_PER_W,), jnp.int32),
        ],
    )(_sc_top2)
    v1, v2, i1, i2 = sc_call(logits)
    vals = jnp.stack([v1, v2], axis=1)
    idx = jnp.stack([i1, i2], axis=1)
    return (vals, idx)


# speedup vs baseline: 1.5242x; 1.0073x over previous
import functools

import jax
import jax.numpy as jnp
from jax import lax
from jax.experimental import pallas as pl

N_TOKENS = 32768
DIM_IN = 4096
NUM_EXPERTS = 64
BT = 1024


def _router_block(x_ref, wt_ref, b_ref, vals_ref, idx_ref):
    logits = jnp.dot(x_ref[...], wt_ref[...], preferred_element_type=jnp.float32)
    logits = logits + b_ref[...]
    iota = lax.broadcasted_iota(jnp.int32, logits.shape, 1)
    max1 = jnp.max(logits, axis=1, keepdims=True)
    idx1 = jnp.min(jnp.where(logits == max1, iota, NUM_EXPERTS), axis=1, keepdims=True)
    masked = jnp.where(iota == idx1, -jnp.inf, logits)
    max2 = jnp.max(masked, axis=1, keepdims=True)
    idx2 = jnp.min(jnp.where(masked == max2, iota, NUM_EXPERTS), axis=1, keepdims=True)
    vals_ref[...] = jnp.concatenate([max1, max2], axis=1)
    idx_ref[...] = jnp.concatenate([idx1, idx2], axis=1)


@jax.jit
def kernel(x, W, b):
    wt = W.T
    b2 = b.reshape(1, NUM_EXPERTS)
    grid = (N_TOKENS // BT,)
    vals, idx = pl.pallas_call(
        _router_block,
        grid=grid,
        in_specs=[
            pl.BlockSpec((BT, DIM_IN), lambda i: (i, 0)),
            pl.BlockSpec((DIM_IN, NUM_EXPERTS), lambda i: (0, 0)),
            pl.BlockSpec((1, NUM_EXPERTS), lambda i: (0, 0)),
        ],
        out_specs=[
            pl.BlockSpec((BT, 2), lambda i: (i, 0)),
            pl.BlockSpec((BT, 2), lambda i: (i, 0)),
        ],
        out_shape=[
            jax.ShapeDtypeStruct((N_TOKENS, 2), jnp.float32),
            jax.ShapeDtypeStruct((N_TOKENS, 2), jnp.int32),
        ],
    )(x, wt, b2)
    return (vals, idx)


# speedup vs baseline: 1.5471x; 1.0150x over previous
import jax
import jax.numpy as jnp
from jax import lax
from jax.experimental import pallas as pl
from jax.experimental.pallas import tpu as pltpu

N_TOKENS = 32768
DIM_IN = 4096
NUM_EXPERTS = 64
BT = 1024


def _router_block(x_ref, w_ref, b_ref, vals_ref, idx_ref, wt_ref):
    @pl.when(pl.program_id(0) == 0)
    def _():
        wt_ref[...] = w_ref[...].T

    logits = jnp.dot(x_ref[...], wt_ref[...], preferred_element_type=jnp.float32)
    logits = logits + b_ref[...]
    iota = lax.broadcasted_iota(jnp.int32, logits.shape, 1)
    max1 = jnp.max(logits, axis=1, keepdims=True)
    idx1 = jnp.min(jnp.where(logits == max1, iota, NUM_EXPERTS), axis=1, keepdims=True)
    masked = jnp.where(iota == idx1, -jnp.inf, logits)
    max2 = jnp.max(masked, axis=1, keepdims=True)
    idx2 = jnp.min(jnp.where(masked == max2, iota, NUM_EXPERTS), axis=1, keepdims=True)
    vals_ref[...] = jnp.concatenate([max1, max2], axis=1)
    idx_ref[...] = jnp.concatenate([idx1, idx2], axis=1)


@jax.jit
def kernel(x, W, b):
    b2 = b.reshape(1, NUM_EXPERTS)
    vals, idx = pl.pallas_call(
        _router_block,
        grid=(N_TOKENS // BT,),
        in_specs=[
            pl.BlockSpec((BT, DIM_IN), lambda i: (i, 0)),
            pl.BlockSpec((NUM_EXPERTS, DIM_IN), lambda i: (0, 0)),
            pl.BlockSpec((1, NUM_EXPERTS), lambda i: (0, 0)),
        ],
        out_specs=[
            pl.BlockSpec((BT, 2), lambda i: (i, 0)),
            pl.BlockSpec((BT, 2), lambda i: (i, 0)),
        ],
        out_shape=[
            jax.ShapeDtypeStruct((N_TOKENS, 2), jnp.float32),
            jax.ShapeDtypeStruct((N_TOKENS, 2), jnp.int32),
        ],
        scratch_shapes=[pltpu.VMEM((DIM_IN, NUM_EXPERTS), jnp.float32)],
    )(x, W, b2)
    return (vals, idx)
